# baseline (speedup 1.0000x reference)
import jax
import jax.numpy as jnp
from jax import lax
from jax.experimental import pallas as pl
from jax.experimental.pallas import tpu as pltpu
from jax.experimental.pallas import tpu_sc as plsc

_SEQ = 4
_DIM = 512
_BATCH = 16384
_NC, _NS = 2, 16
_NW = _NC * _NS
_BCHUNK = 16
_PER_W = _BATCH // _NW
_NDMA = _PER_W // _BCHUNK


def _sc_body(table_hbm, out_hbm, buf, spmem, sem):
    cid = lax.axis_index("c")
    sid = lax.axis_index("s")

    srow = sid * _BCHUNK
    pltpu.sync_copy(table_hbm, buf.at[0])
    n = 1
    while n < _BCHUNK:
        pltpu.sync_copy(buf.at[pl.ds(0, n)], spmem.at[pl.ds(srow + n, n)])
        pltpu.sync_copy(spmem.at[pl.ds(srow + n, n)], buf.at[pl.ds(n, n)])
        n *= 2

    wid = sid * _NC + cid
    copies = [
        pltpu.async_copy(
            buf,
            out_hbm.at[pl.ds((j * _NW + wid) * _BCHUNK, _BCHUNK)],
            sem,
        )
        for j in range(_NDMA)
    ]
    for cp in copies:
        cp.wait()


def kernel(sequence_length, table):
    batch = sequence_length.shape[0]
    mesh = plsc.VectorSubcoreMesh(core_axis_name="c", subcore_axis_name="s")
    sc_fill = pl.kernel(
        _sc_body,
        mesh=mesh,
        out_type=jax.ShapeDtypeStruct((batch, _SEQ, _DIM), jnp.float32),
        scratch_types=[
            pltpu.VMEM((_BCHUNK, _SEQ, _DIM), jnp.float32),
            pltpu.VMEM_SHARED((_NS * _BCHUNK, _SEQ, _DIM), jnp.float32),
            pltpu.SemaphoreType.DMA,
        ],
    )
    return sc_fill(table)


# speedup vs baseline: 1.0034x; 1.0034x over previous
import jax
import jax.numpy as jnp
from jax import lax
from jax.experimental import pallas as pl
from jax.experimental.pallas import tpu as pltpu
from jax.experimental.pallas import tpu_sc as plsc

_SEQ = 4
_DIM = 512
_BATCH = 16384
_NC, _NS = 2, 16
_NW = _NC * _NS
_BCHUNK = 8
_PER_W = _BATCH // _NW
_NDMA = _PER_W // _BCHUNK


def _sc_body(table_hbm, out_hbm, buf, spmem, sem):
    cid = lax.axis_index("c")
    sid = lax.axis_index("s")

    srow = sid * _BCHUNK
    pltpu.sync_copy(table_hbm, buf.at[0])
    n = 1
    while n < _BCHUNK:
        pltpu.sync_copy(buf.at[pl.ds(0, n)], spmem.at[pl.ds(srow + n, n)])
        pltpu.sync_copy(spmem.at[pl.ds(srow + n, n)], buf.at[pl.ds(n, n)])
        n *= 2

    pltpu.sync_copy(buf, spmem.at[pl.ds(srow, _BCHUNK)])

    wid = sid * _NC + cid
    copies = [
        pltpu.async_copy(
            spmem.at[pl.ds(srow, _BCHUNK)] if j % 5 < 2 else buf,
            out_hbm.at[pl.ds((j * _NW + wid) * _BCHUNK, _BCHUNK)],
            sem,
        )
        for j in range(_NDMA)
    ]
    for cp in copies:
        cp.wait()


def kernel(sequence_length, table):
    batch = sequence_length.shape[0]
    mesh = plsc.VectorSubcoreMesh(core_axis_name="c", subcore_axis_name="s")
    sc_fill = pl.kernel(
        _sc_body,
        mesh=mesh,
        out_type=jax.ShapeDtypeStruct((batch, _SEQ, _DIM), jnp.float32),
        scratch_types=[
            pltpu.VMEM((_BCHUNK, _SEQ, _DIM), jnp.float32),
            pltpu.VMEM_SHARED((_NS * _BCHUNK, _SEQ, _DIM), jnp.float32),
            pltpu.SemaphoreType.DMA,
        ],
    )
    return sc_fill(table)


# speedup vs baseline: 1.0072x; 1.0038x over previous
import jax
import jax.numpy as jnp
from jax import lax
from jax.experimental import pallas as pl
from jax.experimental.pallas import tpu as pltpu
from jax.experimental.pallas import tpu_sc as plsc

_SEQ = 4
_DIM = 512
_BATCH = 16384
_NC, _NS = 2, 16
_NW = _NC * _NS
_BCHUNK = 8
_PER_W = _BATCH // _NW
_NDMA = _PER_W // _BCHUNK


def _sc_body(table_hbm, out_hbm, buf, spmem, sem):
    cid = lax.axis_index("c")
    sid = lax.axis_index("s")

    srow = sid * _BCHUNK
    pltpu.sync_copy(table_hbm, buf.at[0])
    n = 1
    while n < _BCHUNK:
        pltpu.sync_copy(buf.at[pl.ds(0, n)], spmem.at[pl.ds(srow + n, n)])
        pltpu.sync_copy(spmem.at[pl.ds(srow + n, n)], buf.at[pl.ds(n, n)])
        n *= 2

    wid = sid * _NC + cid
    copies = [
        pltpu.async_copy(
            buf,
            out_hbm.at[pl.ds((j * _NW + wid) * _BCHUNK, _BCHUNK)],
            sem,
        )
        for j in range(_NDMA)
    ]
    for cp in copies:
        cp.wait()


def kernel(sequence_length, table):
    batch = sequence_length.shape[0]
    mesh = plsc.VectorSubcoreMesh(core_axis_name="c", subcore_axis_name="s")
    sc_fill = pl.kernel(
        _sc_body,
        mesh=mesh,
        out_type=jax.ShapeDtypeStruct((batch, _SEQ, _DIM), jnp.float32),
        scratch_types=[
            pltpu.VMEM((_BCHUNK, _SEQ, _DIM), jnp.float32),
            pltpu.VMEM_SHARED((_NS * _BCHUNK, _SEQ, _DIM), jnp.float32),
            pltpu.SemaphoreType.DMA,
        ],
    )
    return sc_fill(table)
